# baseline (speedup 1.0000x reference)
import functools

import jax
import jax.numpy as jnp
from jax import lax
from jax.experimental import pallas as pl
from jax.experimental.pallas import tpu as pltpu
from jax.experimental.pallas import tpu_sc as plsc

BATCH = 16384
SEQ = 200
NVAL = 10
DIM = 5
NC = 2
NS = 16
NW = NC * NS
WCOLS = BATCH // NW
NSTRIP = SEQ // 8
TSIZE = NVAL * DIM
TBL = 128

_mesh = plsc.VectorSubcoreMesh(core_axis_name="c", subcore_axis_name="s")


@functools.partial(
    pl.kernel,
    out_type=jax.ShapeDtypeStruct((DIM, SEQ, BATCH), jnp.float32),
    mesh=_mesh,
    compiler_params=pltpu.CompilerParams(needs_layout_passes=False),
    scratch_types=[
        pltpu.VMEM((TBL,), jnp.float32),
        pltpu.VMEM((64,), jnp.float32),
        pltpu.VMEM((16,), jnp.float32),
        pltpu.VMEM((16,), jnp.int32),
        pltpu.VMEM((2, 8, WCOLS), jnp.int32),
        pltpu.VMEM((2, DIM, 8, WCOLS), jnp.float32),
        pltpu.SemaphoreType.DMA((2,)),
        pltpu.SemaphoreType.DMA((2,)),
    ],
)
def _sc_lookup(xt_hbm, w_hbm, out_hbm, tbl_v, wtmp, scale_v, x0tmp, xtile, otile,
               sem_in, sem_out):
    wid = lax.axis_index("s") * NC + lax.axis_index("c")
    lanes = lax.iota(jnp.int32, 16)

    pltpu.sync_copy(w_hbm, wtmp)
    pltpu.sync_copy(xt_hbm.at[0, pl.ds(0, 16)], x0tmp)

    x16 = x0tmp[...]
    x00 = jnp.sum(jnp.where(lanes == 0, x16, 0))
    waddr = jnp.minimum(x00 * DIM + lanes, TSIZE - 1)
    row = plsc.load_gather(wtmp, [waddr])
    valid = lanes < DIM
    rowm = jnp.where(valid, row, jnp.float32(-1e30))
    m = jnp.max(rowm)
    e = jnp.where(valid, jnp.exp(rowm - m), jnp.float32(0.0))
    scale_v[...] = e / jnp.sum(e)

    for k in range(TBL // 16):
        j = lanes + k * 16
        jm = jnp.where(j < TSIZE, j, j - TSIZE)
        jm = jnp.minimum(jm, TSIZE - 1)
        wv = plsc.load_gather(wtmp, [jm])
        sv = plsc.load_gather(scale_v, [jm % DIM])
        tbl_v[pl.ds(k * 16, 16)] = jnp.where(j < TSIZE, wv, wv * sv)

    luts = []
    for c in range(DIM):
        laddr = jnp.minimum(lanes * DIM + c, TSIZE - 1)
        luts.append(plsc.load_gather(wtmp, [laddr]))

    col0 = wid * WCOLS

    def in_copy(t, b):
        return pltpu.make_async_copy(
            xt_hbm.at[pl.ds(t * 8, 8), pl.ds(col0, WCOLS)],
            xtile.at[b], sem_in.at[b])

    def out_copy(t, b):
        return pltpu.make_async_copy(
            otile.at[b], out_hbm.at[:, pl.ds(t * 8, 8), pl.ds(col0, WCOLS)],
            sem_out.at[b])

    in_copy(0, 0).start()

    def strip_body(t, carry):
        b = t & 1

        @pl.when(t + 1 < NSTRIP)
        def _prefetch():
            in_copy(t + 1, 1 - b).start()

        in_copy(t, b).wait()

        @pl.when(t >= 2)
        def _drain_prev():
            out_copy(t - 2, b).wait()

        row0_extra = jnp.where(t == 0, TSIZE, 0)

        @plsc.parallel_loop(0, WCOLS // 16, unroll=4)
        def vec_body(i):
            o = i * 16
            for r in range(8):
                idx = xtile[b, r, pl.ds(o, 16)]
                if r == 0:
                    addr = idx * DIM + row0_extra
                    for c in range(DIM):
                        otile[b, c, r, pl.ds(o, 16)] = plsc.load_gather(
                            tbl_v, [addr + c])
                else:
                    for c in range(DIM):
                        otile[b, c, r, pl.ds(o, 16)] = luts[c].at[idx].get(
                            mode="promise_in_bounds")

        out_copy(t, b).start()
        return carry

    lax.fori_loop(0, NSTRIP, strip_body, 0)
    out_copy(NSTRIP - 2, 1).wait()
    out_copy(NSTRIP - 1, 0).wait()


def kernel(x, W):
    xt = x.T
    wf = jnp.pad(W.reshape(-1), (0, 64 - TSIZE))
    out = _sc_lookup(xt, wf)
    return out.transpose(2, 1, 0)
